# baseline (speedup 1.0000x reference)
import jax
import jax.numpy as jnp
from jax import lax
from jax.experimental import pallas as pl
from jax.experimental.pallas import tpu as pltpu
from jax.experimental.pallas import tpu_sc as plsc

NUM_CLASS = 100000
CTX_DIM = 512
N_CLS_CTX = 8
BATCH = 1024
SEQ_LEN = 77

NUM_CORES = 2
NUM_SUBCORES = 16
NUM_WORKERS = NUM_CORES * NUM_SUBCORES
BPW = BATCH // NUM_WORKERS
K = 8
NCHUNK = BPW // K
NTAIL = SEQ_LEN - 16
LANES = CTX_DIM // 16


def _sc_body(label_h, cls_h, pre_h, suf_h, fin_h, out_h, idx_v, head_v, z_v,
             rows_v, fix_v, gsem, hsem, zsem):
    wid = lax.axis_index("s") * NUM_CORES + lax.axis_index("c")
    base = wid * BPW
    pltpu.sync_copy(label_h.at[pl.ds(base, BPW)], idx_v)
    g_pending = pltpu.async_copy(
        cls_h.at[idx_v.at[pl.ds(0, K)]], rows_v, gsem)
    f1 = pltpu.async_copy(pre_h.at[0], fix_v.at[pl.ds(0, 4)], hsem)
    f2 = pltpu.async_copy(suf_h.at[0], fix_v.at[pl.ds(4, 3)], hsem)
    f3 = pltpu.async_copy(fin_h.at[0], fix_v.at[pl.ds(7, 1)], hsem)

    zero16 = jnp.zeros((16,), jnp.float32)

    @plsc.parallel_loop(0, BPW * LANES, unroll=4)
    def _zfill(i):
        z_v[i // LANES, pl.ds((i % LANES) * 16, 16)] = zero16

    def z_issue(lo, n):
        def zb(r, _):
            pltpu.async_copy(z_v, out_h.at[16 + r, pl.ds(base, BPW)], zsem)
            return 0
        lax.fori_loop(lo, lo + n, zb, 0)

    ZSCHED = (13, 16, 16, 16, 0)

    f1.wait()
    f2.wait()
    f3.wait()

    @plsc.parallel_loop(0, 8 * K * LANES, unroll=4)
    def _hfill(i):
        rf = i // (K * LANES)
        rem = i % (K * LANES)
        e = rem // LANES
        col = (rem % LANES) * 16
        ro = jnp.where(rf < 4, rf, jnp.where(rf < 7, rf + 4, 15))
        head_v[ro, e, pl.ds(col, 16)] = fix_v[rf, pl.ds(col, 16)]

    z_issue(0, ZSCHED[0])
    zoff = ZSCHED[0]
    h_pending = None
    for c in range(NCHUNK):
        g_pending.wait()
        if h_pending is not None:
            h_pending.wait()

        @plsc.parallel_loop(0, K * 4 * LANES, unroll=4)
        def _asm(i):
            e = i // (4 * LANES)
            rem = i % (4 * LANES)
            k = rem // LANES
            col = (rem % LANES) * 16
            head_v[4 + k, e, pl.ds(col, 16)] = rows_v[e, k, pl.ds(col, 16)]
            head_v[11 + k, e, pl.ds(col, 16)] = rows_v[e, 4 + k, pl.ds(col, 16)]

        if c + 1 < NCHUNK:
            g_pending = pltpu.async_copy(
                cls_h.at[idx_v.at[pl.ds((c + 1) * K, K)]], rows_v, gsem)
        h_pending = pltpu.async_copy(
            head_v, out_h.at[pl.ds(0, 16), pl.ds(base + c * K, K)], hsem)
        if ZSCHED[1 + c]:
            z_issue(zoff, ZSCHED[1 + c])
            zoff += ZSCHED[1 + c]
    h_pending.wait()

    def z_drain(i, _):
        pltpu.make_async_copy(
            z_v, out_h.at[16, pl.ds(base, BPW)], zsem).wait()
        return 0
    lax.fori_loop(0, NTAIL, z_drain, 0)


@jax.jit
def _sc_prompts(label, cls_ctx, prefix, suffix, final_suffix):
    mesh = plsc.VectorSubcoreMesh(core_axis_name="c", subcore_axis_name="s")
    return pl.kernel(
        _sc_body,
        out_type=jax.ShapeDtypeStruct((SEQ_LEN, BATCH, CTX_DIM), jnp.float32),
        mesh=mesh,
        scratch_types=[
            pltpu.VMEM((BPW,), jnp.int32),
            pltpu.VMEM((16, K, CTX_DIM), jnp.float32),
            pltpu.VMEM((BPW, CTX_DIM), jnp.float32),
            pltpu.VMEM((K, N_CLS_CTX, CTX_DIM), jnp.float32),
            pltpu.VMEM((8, CTX_DIM), jnp.float32),
            pltpu.SemaphoreType.DMA,
            pltpu.SemaphoreType.DMA,
            pltpu.SemaphoreType.DMA,
        ],
    )(label, cls_ctx, prefix, suffix, final_suffix)


def kernel(label, cls_ctx, prefix, suffix, final_suffix):
    out_t = _sc_prompts(label.astype(jnp.int32), cls_ctx, prefix, suffix,
                        final_suffix)
    return out_t.transpose(1, 0, 2)


# speedup vs baseline: 1.0096x; 1.0096x over previous
import jax
import jax.numpy as jnp
from jax import lax
from jax.experimental import pallas as pl
from jax.experimental.pallas import tpu as pltpu
from jax.experimental.pallas import tpu_sc as plsc

NUM_CLASS = 100000
CTX_DIM = 512
N_CLS_CTX = 8
BATCH = 1024
SEQ_LEN = 77

NUM_CORES = 2
NUM_SUBCORES = 16
NUM_WORKERS = NUM_CORES * NUM_SUBCORES
BPW = BATCH // NUM_WORKERS
K = 8
NCHUNK = BPW // K
NTAIL = SEQ_LEN - 16
LANES = CTX_DIM // 16


def _sc_body(label_h, cls_h, pre_h, suf_h, fin_h, out_h, idx_v, head_v, z_v,
             rows_v, fix_v, gsem, hsem, zsem):
    wid = lax.axis_index("s") * NUM_CORES + lax.axis_index("c")
    base = wid * BPW
    pltpu.sync_copy(label_h.at[pl.ds(base, BPW)], idx_v)
    g_pending = pltpu.async_copy(
        cls_h.at[idx_v.at[pl.ds(0, K)]], rows_v, gsem)
    f1 = pltpu.async_copy(pre_h.at[0], fix_v.at[pl.ds(0, 4)], hsem)
    f2 = pltpu.async_copy(suf_h.at[0], fix_v.at[pl.ds(4, 3)], hsem)
    f3 = pltpu.async_copy(fin_h.at[0], fix_v.at[pl.ds(7, 1)], hsem)

    zero16 = jnp.zeros((16,), jnp.float32)

    @plsc.parallel_loop(0, BPW * LANES, unroll=4)
    def _zfill(i):
        z_v[i // LANES, pl.ds((i % LANES) * 16, 16)] = zero16

    def z_issue(lo, n):
        def zb(r, _):
            pltpu.async_copy(z_v, out_h.at[16 + r, pl.ds(base, BPW)], zsem)
            return 0
        lax.fori_loop(lo, lo + n, zb, 0)

    ZSCHED = (9, 13, 13, 13, 13)

    f1.wait()
    f2.wait()
    f3.wait()

    @plsc.parallel_loop(0, 8 * K * LANES, unroll=4)
    def _hfill(i):
        rf = i // (K * LANES)
        rem = i % (K * LANES)
        e = rem // LANES
        col = (rem % LANES) * 16
        ro = jnp.where(rf < 4, rf, jnp.where(rf < 7, rf + 4, 15))
        head_v[ro, e, pl.ds(col, 16)] = fix_v[rf, pl.ds(col, 16)]

    z_issue(0, ZSCHED[0])
    zoff = ZSCHED[0]
    h_pending = None
    for c in range(NCHUNK):
        g_pending.wait()
        if h_pending is not None:
            h_pending.wait()

        @plsc.parallel_loop(0, K * 4 * LANES, unroll=4)
        def _asm(i):
            e = i // (4 * LANES)
            rem = i % (4 * LANES)
            k = rem // LANES
            col = (rem % LANES) * 16
            head_v[4 + k, e, pl.ds(col, 16)] = rows_v[e, k, pl.ds(col, 16)]
            head_v[11 + k, e, pl.ds(col, 16)] = rows_v[e, 4 + k, pl.ds(col, 16)]

        if c + 1 < NCHUNK:
            g_pending = pltpu.async_copy(
                cls_h.at[idx_v.at[pl.ds((c + 1) * K, K)]], rows_v, gsem)
        h_pending = pltpu.async_copy(
            head_v, out_h.at[pl.ds(0, 16), pl.ds(base + c * K, K)], hsem)
        if ZSCHED[1 + c]:
            z_issue(zoff, ZSCHED[1 + c])
            zoff += ZSCHED[1 + c]
    h_pending.wait()

    def z_drain(i, _):
        pltpu.make_async_copy(
            z_v, out_h.at[16, pl.ds(base, BPW)], zsem).wait()
        return 0
    lax.fori_loop(0, NTAIL, z_drain, 0)


@jax.jit
def _sc_prompts(label, cls_ctx, prefix, suffix, final_suffix):
    mesh = plsc.VectorSubcoreMesh(core_axis_name="c", subcore_axis_name="s")
    return pl.kernel(
        _sc_body,
        out_type=jax.ShapeDtypeStruct((SEQ_LEN, BATCH, CTX_DIM), jnp.float32),
        mesh=mesh,
        scratch_types=[
            pltpu.VMEM((BPW,), jnp.int32),
            pltpu.VMEM((16, K, CTX_DIM), jnp.float32),
            pltpu.VMEM((BPW, CTX_DIM), jnp.float32),
            pltpu.VMEM((K, N_CLS_CTX, CTX_DIM), jnp.float32),
            pltpu.VMEM((8, CTX_DIM), jnp.float32),
            pltpu.SemaphoreType.DMA,
            pltpu.SemaphoreType.DMA,
            pltpu.SemaphoreType.DMA,
        ],
    )(label, cls_ctx, prefix, suffix, final_suffix)


def kernel(label, cls_ctx, prefix, suffix, final_suffix):
    out_t = _sc_prompts(label.astype(jnp.int32), cls_ctx, prefix, suffix,
                        final_suffix)
    return out_t.transpose(1, 0, 2)


# speedup vs baseline: 1.0099x; 1.0002x over previous
import jax
import jax.numpy as jnp
from jax import lax
from jax.experimental import pallas as pl
from jax.experimental.pallas import tpu as pltpu
from jax.experimental.pallas import tpu_sc as plsc

NUM_CLASS = 100000
CTX_DIM = 512
N_CLS_CTX = 8
BATCH = 1024
SEQ_LEN = 77

NUM_CORES = 2
NUM_SUBCORES = 16
NUM_WORKERS = NUM_CORES * NUM_SUBCORES
BPW = BATCH // NUM_WORKERS
K = 8
NCHUNK = BPW // K
NTAIL = SEQ_LEN - 16
LANES = CTX_DIM // 16


def _sc_body(label_h, cls_h, pre_h, suf_h, fin_h, out_h, idx_v, head_v, z_v,
             rows_v, fix_v, gsem, hsem, zsem):
    wid = lax.axis_index("s") * NUM_CORES + lax.axis_index("c")
    base = wid * BPW
    pltpu.sync_copy(label_h.at[pl.ds(base, BPW)], idx_v)
    g_pending = pltpu.async_copy(
        cls_h.at[idx_v.at[pl.ds(0, K)]], rows_v, gsem)
    f1 = pltpu.async_copy(pre_h.at[0], fix_v.at[pl.ds(0, 4)], hsem)
    f2 = pltpu.async_copy(suf_h.at[0], fix_v.at[pl.ds(4, 3)], hsem)
    f3 = pltpu.async_copy(fin_h.at[0], fix_v.at[pl.ds(7, 1)], hsem)

    zero16 = jnp.zeros((16,), jnp.float32)

    @plsc.parallel_loop(0, BPW * LANES, unroll=4)
    def _zfill(i):
        z_v[i // LANES, pl.ds((i % LANES) * 16, 16)] = zero16

    def z_issue(lo, n):
        def zb(r, _):
            pltpu.async_copy(z_v, out_h.at[16 + r, pl.ds(base, BPW)], zsem)
            return 0
        lax.fori_loop(lo, lo + n, zb, 0)

    ZSCHED = (5, 14, 14, 14, 14)

    f1.wait()
    f2.wait()
    f3.wait()

    @plsc.parallel_loop(0, 8 * K * LANES, unroll=4)
    def _hfill(i):
        rf = i // (K * LANES)
        rem = i % (K * LANES)
        e = rem // LANES
        col = (rem % LANES) * 16
        ro = jnp.where(rf < 4, rf, jnp.where(rf < 7, rf + 4, 15))
        head_v[ro, e, pl.ds(col, 16)] = fix_v[rf, pl.ds(col, 16)]

    z_issue(0, ZSCHED[0])
    zoff = ZSCHED[0]
    h_pending = None
    for c in range(NCHUNK):
        g_pending.wait()
        if h_pending is not None:
            h_pending.wait()

        @plsc.parallel_loop(0, K * 4 * LANES, unroll=4)
        def _asm(i):
            e = i // (4 * LANES)
            rem = i % (4 * LANES)
            k = rem // LANES
            col = (rem % LANES) * 16
            head_v[4 + k, e, pl.ds(col, 16)] = rows_v[e, k, pl.ds(col, 16)]
            head_v[11 + k, e, pl.ds(col, 16)] = rows_v[e, 4 + k, pl.ds(col, 16)]

        if c + 1 < NCHUNK:
            g_pending = pltpu.async_copy(
                cls_h.at[idx_v.at[pl.ds((c + 1) * K, K)]], rows_v, gsem)
        h_pending = pltpu.async_copy(
            head_v, out_h.at[pl.ds(0, 16), pl.ds(base + c * K, K)], hsem)
        if ZSCHED[1 + c]:
            z_issue(zoff, ZSCHED[1 + c])
            zoff += ZSCHED[1 + c]
    h_pending.wait()

    def z_drain(i, _):
        pltpu.make_async_copy(
            z_v, out_h.at[16, pl.ds(base, BPW)], zsem).wait()
        return 0
    lax.fori_loop(0, NTAIL, z_drain, 0)


@jax.jit
def _sc_prompts(label, cls_ctx, prefix, suffix, final_suffix):
    mesh = plsc.VectorSubcoreMesh(core_axis_name="c", subcore_axis_name="s")
    return pl.kernel(
        _sc_body,
        out_type=jax.ShapeDtypeStruct((SEQ_LEN, BATCH, CTX_DIM), jnp.float32),
        mesh=mesh,
        scratch_types=[
            pltpu.VMEM((BPW,), jnp.int32),
            pltpu.VMEM((16, K, CTX_DIM), jnp.float32),
            pltpu.VMEM((BPW, CTX_DIM), jnp.float32),
            pltpu.VMEM((K, N_CLS_CTX, CTX_DIM), jnp.float32),
            pltpu.VMEM((8, CTX_DIM), jnp.float32),
            pltpu.SemaphoreType.DMA,
            pltpu.SemaphoreType.DMA,
            pltpu.SemaphoreType.DMA,
        ],
    )(label, cls_ctx, prefix, suffix, final_suffix)


def kernel(label, cls_ctx, prefix, suffix, final_suffix):
    out_t = _sc_prompts(label.astype(jnp.int32), cls_ctx, prefix, suffix,
                        final_suffix)
    return out_t.transpose(1, 0, 2)
